# baseline (speedup 1.0000x reference)
import jax
import jax.numpy as jnp
from jax import lax
from jax.experimental import pallas as pl
from jax.experimental.pallas import tpu as pltpu
from jax.experimental.pallas import tpu_sc as plsc

NC = 2
NS = 16


def _index_add(input_tensor, index, source_tensor):
    N, D = input_tensor.shape
    S = index.shape[0]
    HALF = N // NC
    R = 20000
    CHUNKS = HALF // R
    SL = R // NS
    PIECE = SL // 2
    SPT = S // NS
    NV = SPT // 16
    TRASH = SPT + 16
    EROWS = 64

    mesh = plsc.VectorSubcoreMesh(core_axis_name="c", subcore_axis_name="s")

    def body(inp_hbm, idx_hbm, src_hbm, out_hbm,
             spmem, tbuf, idx_v, src_v, floc, fpos, el2d, stage, ramp):
        c = lax.axis_index("c")
        s = lax.axis_index("s")
        pltpu.sync_copy(idx_hbm.at[pl.ds(s * SPT, SPT)], idx_v)
        pltpu.sync_copy(src_hbm.at[pl.ds(s * SPT * D, SPT * D)], src_v)
        ramp[pl.ds(0, 16)] = lax.iota(jnp.int32, 16)

        def chunk_body(ci, _):
            base = c * HALF + ci * R
            lo = base
            hi = base + R
            ramp16 = ramp[pl.ds(0, 16)]

            for h in range(2):
                off = (base + s * SL + h * PIECE) * D
                pltpu.sync_copy(inp_hbm.at[pl.ds(off, PIECE * D)], tbuf)
                pltpu.sync_copy(
                    tbuf, spmem.at[pl.ds((s * SL + h * PIECE) * D, PIECE * D)])
            plsc.subcore_barrier()

            def scan(i, cnt):
                v = idx_v[pl.ds(i * 16, 16)]
                m = (v >= lo) & (v < hi)
                pc = plsc.cumsum(m.astype(jnp.int32))
                p = jnp.where(m, cnt + pc - 1, TRASH)
                plsc.store_scatter(floc, [p], v - lo)
                plsc.store_scatter(fpos, [p], ramp16 + i * 16)
                return cnt + pc[15]
            cnt = lax.fori_loop(0, NV, scan, jnp.int32(0))

            dummy16 = jnp.full((16,), R + s, jnp.int32)
            plsc.store_scatter(floc, [cnt + ramp16], dummy16)
            plsc.store_scatter(fpos, [cnt + ramp16],
                               jnp.zeros((16,), jnp.int32))

            def block(b, _):
                rem = cnt - b * 256

                def expand(gl, _):
                    gg = b * 16 + gl
                    rv = floc[pl.ds(gg * 16, 16)]
                    pv = fpos[pl.ds(gg * 16, 16)]
                    for jj in range(D):
                        gv = plsc.load_gather(src_v, [pv * D + jj])
                        q = ramp16 * D + jj
                        rowv = gl * 4 + (q >> 7)
                        colv = q & 127
                        plsc.store_scatter(stage, [rowv, colv], gv)
                        plsc.store_scatter(el2d, [rowv, colv], rv * D + jj)
                    return 0
                lax.fori_loop(0, jnp.clip((rem + 15) // 16, 0, 16), expand, 0)

                def apply(d, _):
                    pltpu.sync_copy(stage.at[d], spmem.at[el2d.at[d]],
                                    add=True)
                    return 0
                lax.fori_loop(0, jnp.clip((rem + 3) // 4, 0, 64), apply, 0)
                return 0
            lax.fori_loop(0, (cnt + 255) // 256, block, 0)
            plsc.subcore_barrier()

            for h in range(2):
                off = (base + s * SL + h * PIECE) * D
                pltpu.sync_copy(
                    spmem.at[pl.ds((s * SL + h * PIECE) * D, PIECE * D)], tbuf)
                pltpu.sync_copy(tbuf, out_hbm.at[pl.ds(off, PIECE * D)])
            return 0

        lax.fori_loop(0, CHUNKS, chunk_body, 0)

    out = pl.kernel(
        body,
        out_type=jax.ShapeDtypeStruct((N * D,), jnp.float32),
        mesh=mesh,
        compiler_params=pltpu.CompilerParams(needs_layout_passes=False),
        scratch_types=[
            pltpu.VMEM_SHARED(((R + NS) * D,), jnp.float32),
            pltpu.VMEM((PIECE * D,), jnp.float32),
            pltpu.VMEM((SPT,), jnp.int32),
            pltpu.VMEM((SPT * D,), jnp.float32),
            pltpu.VMEM((SPT + 32,), jnp.int32),
            pltpu.VMEM((SPT + 32,), jnp.int32),
            pltpu.VMEM((EROWS, 128), jnp.int32),
            pltpu.VMEM((EROWS, 128), jnp.float32),
            pltpu.VMEM((16,), jnp.int32),
        ],
    )(input_tensor.reshape(N * D), index.astype(jnp.int32),
      source_tensor.reshape(S * D))
    return out.reshape(N, D)


def kernel(input_tensor, index, source_tensor):
    return _index_add(input_tensor, index, source_tensor)


# speedup vs baseline: 1.0345x; 1.0345x over previous
import jax
import jax.numpy as jnp
from jax import lax
from jax.experimental import pallas as pl
from jax.experimental.pallas import tpu as pltpu
from jax.experimental.pallas import tpu_sc as plsc

NC = 2
NS = 16


def _index_add(input_tensor, index, source_tensor):
    N, D = input_tensor.shape
    S = index.shape[0]
    HALF = N // NC
    R = 20000
    CHUNKS = HALF // R
    SL = R // NS
    PIECE = SL // 2
    SPT = S // NS
    NV = SPT // 16
    TRASH = SPT + 16
    EROWS = 48

    mesh = plsc.VectorSubcoreMesh(core_axis_name="c", subcore_axis_name="s")

    def body(inp_hbm, idx_hbm, src_hbm, out_hbm,
             spmem, tbufa, tbufb, idx_v, src_v, floc, fpos, el2d, stage,
             ramp, sema, semb):
        c = lax.axis_index("c")
        s = lax.axis_index("s")
        pltpu.sync_copy(idx_hbm.at[pl.ds(s * SPT, SPT)], idx_v)
        pltpu.sync_copy(src_hbm.at[pl.ds(s * SPT * D, SPT * D)], src_v)
        ramp[pl.ds(0, 16)] = lax.iota(jnp.int32, 16)

        def chunk_body(ci, _):
            base = c * HALF + ci * R
            lo = base
            hi = base + R
            ramp16 = ramp[pl.ds(0, 16)]

            off0 = (base + s * SL) * D
            off1 = off0 + PIECE * D
            da = pltpu.async_copy(inp_hbm.at[pl.ds(off0, PIECE * D)],
                                  tbufa, sema)
            db = pltpu.async_copy(inp_hbm.at[pl.ds(off1, PIECE * D)],
                                  tbufb, semb)

            def scan(i, cnt):
                v = idx_v[pl.ds(i * 16, 16)]
                m = (v >= lo) & (v < hi)
                pc = plsc.cumsum(m.astype(jnp.int32))
                p = jnp.where(m, cnt + pc - 1, TRASH)
                plsc.store_scatter(floc, [p], v - lo)
                plsc.store_scatter(fpos, [p], ramp16 + i * 16)
                return cnt + pc[15]
            cnt = lax.fori_loop(0, NV, scan, jnp.int32(0))

            dummy16 = jnp.full((16,), R + s, jnp.int32)
            plsc.store_scatter(floc, [cnt + ramp16], dummy16)
            plsc.store_scatter(fpos, [cnt + ramp16],
                               jnp.zeros((16,), jnp.int32))

            da.wait()
            da2 = pltpu.async_copy(
                tbufa, spmem.at[pl.ds((s * SL) * D, PIECE * D)], sema)
            db.wait()
            db2 = pltpu.async_copy(
                tbufb, spmem.at[pl.ds((s * SL + PIECE) * D, PIECE * D)], semb)
            da2.wait()
            db2.wait()
            plsc.subcore_barrier()

            def block(b, _):
                rem = cnt - b * 192

                def expand(gl, _):
                    gg = b * 12 + gl
                    rv = floc[pl.ds(gg * 16, 16)]
                    pv = fpos[pl.ds(gg * 16, 16)]
                    for jj in range(D):
                        gv = plsc.load_gather(src_v, [pv * D + jj])
                        q = ramp16 * D + jj
                        rowv = gl * 4 + (q >> 7)
                        colv = q & 127
                        plsc.store_scatter(stage, [rowv, colv], gv)
                        plsc.store_scatter(el2d, [rowv, colv], rv * D + jj)
                    return 0
                lax.fori_loop(0, jnp.clip((rem + 15) // 16, 0, 12), expand, 0)

                def apply(d, _):
                    pltpu.sync_copy(stage.at[d], spmem.at[el2d.at[d]],
                                    add=True)
                    return 0
                lax.fori_loop(0, jnp.clip((rem + 3) // 4, 0, 48), apply, 0)
                return 0
            lax.fori_loop(0, (cnt + 191) // 192, block, 0)
            plsc.subcore_barrier()

            oa = pltpu.async_copy(
                spmem.at[pl.ds((s * SL) * D, PIECE * D)], tbufa, sema)
            ob = pltpu.async_copy(
                spmem.at[pl.ds((s * SL + PIECE) * D, PIECE * D)], tbufb, semb)
            oa.wait()
            oa2 = pltpu.async_copy(tbufa, out_hbm.at[pl.ds(off0, PIECE * D)],
                                   sema)
            ob.wait()
            ob2 = pltpu.async_copy(tbufb, out_hbm.at[pl.ds(off1, PIECE * D)],
                                   semb)
            oa2.wait()
            ob2.wait()
            return 0

        lax.fori_loop(0, CHUNKS, chunk_body, 0)

    out = pl.kernel(
        body,
        out_type=jax.ShapeDtypeStruct((N * D,), jnp.float32),
        mesh=mesh,
        compiler_params=pltpu.CompilerParams(needs_layout_passes=False),
        scratch_types=[
            pltpu.VMEM_SHARED(((R + NS) * D,), jnp.float32),
            pltpu.VMEM((PIECE * D,), jnp.float32),
            pltpu.VMEM((PIECE * D,), jnp.float32),
            pltpu.VMEM((SPT,), jnp.int32),
            pltpu.VMEM((SPT * D,), jnp.float32),
            pltpu.VMEM((SPT + 32,), jnp.int32),
            pltpu.VMEM((SPT + 32,), jnp.int32),
            pltpu.VMEM((EROWS, 128), jnp.int32),
            pltpu.VMEM((EROWS, 128), jnp.float32),
            pltpu.VMEM((16,), jnp.int32),
            pltpu.SemaphoreType.DMA,
            pltpu.SemaphoreType.DMA,
        ],
    )(input_tensor.reshape(N * D), index.astype(jnp.int32),
      source_tensor.reshape(S * D))
    return out.reshape(N, D)


def kernel(input_tensor, index, source_tensor):
    return _index_add(input_tensor, index, source_tensor)


# speedup vs baseline: 1.0674x; 1.0318x over previous
import jax
import jax.numpy as jnp
from jax import lax
from jax.experimental import pallas as pl
from jax.experimental.pallas import tpu as pltpu
from jax.experimental.pallas import tpu_sc as plsc

NC = 2
NS = 16


def _index_add(input_tensor, index, source_tensor):
    N, D = input_tensor.shape
    S = index.shape[0]
    HALF = N // NC
    R = 20000
    CHUNKS = HALF // R
    SL = R // NS
    PIECE = SL // 2
    SPT = S // NS
    NV = SPT // 16
    TRASH = SPT + 16
    EROWS = 48

    mesh = plsc.VectorSubcoreMesh(core_axis_name="c", subcore_axis_name="s")

    def body(inp_hbm, idx_hbm, src_hbm, out_hbm,
             spmem, tbufa, tbufb, idx_v, src_v, floc, fpos, el2d, stage,
             ramp, sema, semb, semc):
        c = lax.axis_index("c")
        s = lax.axis_index("s")
        pltpu.sync_copy(idx_hbm.at[pl.ds(s * SPT, SPT)], idx_v)
        pltpu.sync_copy(src_hbm.at[pl.ds(s * SPT * D, SPT * D)], src_v)
        ramp[pl.ds(0, 16)] = lax.iota(jnp.int32, 16)

        def chunk_body(ci, _):
            base = c * HALF + ci * R
            lo = base
            hi = base + R
            ramp16 = ramp[pl.ds(0, 16)]

            off0 = (base + s * SL) * D
            off1 = off0 + PIECE * D
            da = pltpu.async_copy(inp_hbm.at[pl.ds(off0, PIECE * D)],
                                  tbufa, sema)
            db = pltpu.async_copy(inp_hbm.at[pl.ds(off1, PIECE * D)],
                                  tbufb, semb)

            def scan(i, cnt):
                v = idx_v[pl.ds(i * 16, 16)]
                m = (v >= lo) & (v < hi)
                pc = plsc.cumsum(m.astype(jnp.int32))
                p = jnp.where(m, cnt + pc - 1, TRASH)
                plsc.store_scatter(floc, [p], v - lo)
                plsc.store_scatter(fpos, [p], ramp16 + i * 16)
                return cnt + pc[15]
            cnt = lax.fori_loop(0, NV, scan, jnp.int32(0))

            dummy16 = jnp.full((16,), R + s, jnp.int32)
            plsc.store_scatter(floc, [cnt + ramp16], dummy16)
            plsc.store_scatter(fpos, [cnt + ramp16],
                               jnp.zeros((16,), jnp.int32))

            def expand_block(b):
                rem = cnt - b * 192

                def expand(gl, _):
                    gg = b * 12 + gl
                    rv = floc[pl.ds(gg * 16, 16)]
                    pv = fpos[pl.ds(gg * 16, 16)]
                    for jj in range(D):
                        gv = plsc.load_gather(src_v, [pv * D + jj])
                        q = ramp16 * D + jj
                        rowv = gl * 4 + (q >> 7)
                        colv = q & 127
                        plsc.store_scatter(stage, [rowv, colv], gv)
                        plsc.store_scatter(el2d, [rowv, colv], rv * D + jj)
                    return 0
                lax.fori_loop(0, jnp.clip((rem + 15) // 16, 0, 12), expand, 0)
                return rem

            def apply_block(rem):
                dcnt = jnp.clip((rem + 3) // 4, 0, 48)

                def fire(d, _):
                    pltpu.async_copy(stage.at[d], spmem.at[el2d.at[d]], semc,
                                     add=True)
                    return 0
                lax.fori_loop(0, dcnt, fire, 0)

                def drain(d, _):
                    pltpu.make_async_copy(stage.at[d], spmem.at[el2d.at[d]],
                                          semc).wait()
                    return 0
                lax.fori_loop(0, dcnt, drain, 0)

            rem0 = expand_block(0)

            da.wait()
            da2 = pltpu.async_copy(
                tbufa, spmem.at[pl.ds((s * SL) * D, PIECE * D)], sema)
            db.wait()
            db2 = pltpu.async_copy(
                tbufb, spmem.at[pl.ds((s * SL + PIECE) * D, PIECE * D)], semb)
            da2.wait()
            db2.wait()
            plsc.subcore_barrier()

            apply_block(rem0)

            def extra(b, _):
                apply_block(expand_block(b))
                return 0
            lax.fori_loop(1, (cnt + 191) // 192, extra, 0)
            plsc.subcore_barrier()

            oa = pltpu.async_copy(
                spmem.at[pl.ds((s * SL) * D, PIECE * D)], tbufa, sema)
            ob = pltpu.async_copy(
                spmem.at[pl.ds((s * SL + PIECE) * D, PIECE * D)], tbufb, semb)
            oa.wait()
            oa2 = pltpu.async_copy(tbufa, out_hbm.at[pl.ds(off0, PIECE * D)],
                                   sema)
            ob.wait()
            ob2 = pltpu.async_copy(tbufb, out_hbm.at[pl.ds(off1, PIECE * D)],
                                   semb)
            oa2.wait()
            ob2.wait()
            return 0

        lax.fori_loop(0, CHUNKS, chunk_body, 0)

    out = pl.kernel(
        body,
        out_type=jax.ShapeDtypeStruct((N * D,), jnp.float32),
        mesh=mesh,
        compiler_params=pltpu.CompilerParams(needs_layout_passes=False),
        scratch_types=[
            pltpu.VMEM_SHARED(((R + NS) * D,), jnp.float32),
            pltpu.VMEM((PIECE * D,), jnp.float32),
            pltpu.VMEM((PIECE * D,), jnp.float32),
            pltpu.VMEM((SPT,), jnp.int32),
            pltpu.VMEM((SPT * D,), jnp.float32),
            pltpu.VMEM((SPT + 32,), jnp.int32),
            pltpu.VMEM((SPT + 32,), jnp.int32),
            pltpu.VMEM((EROWS, 128), jnp.int32),
            pltpu.VMEM((EROWS, 128), jnp.float32),
            pltpu.VMEM((16,), jnp.int32),
            pltpu.SemaphoreType.DMA,
            pltpu.SemaphoreType.DMA,
            pltpu.SemaphoreType.DMA,
        ],
    )(input_tensor.reshape(N * D), index.astype(jnp.int32),
      source_tensor.reshape(S * D))
    return out.reshape(N, D)


def kernel(input_tensor, index, source_tensor):
    return _index_add(input_tensor, index, source_tensor)
